# Initial kernel scaffold; baseline (speedup 1.0000x reference)
import jax
import jax.numpy as jnp
from jax.experimental import pallas as pl


def kernel(y_soft, y_true, mask, edge_index):
    raise NotImplementedError("write your pallas kernel here")


# speedup vs baseline: 5.8032x; 5.8032x over previous
import functools

import jax
import jax.numpy as jnp
from jax import lax
from jax.experimental import pallas as pl
from jax.experimental.pallas import tpu as pltpu
from jax.experimental.pallas import tpu_sc as plsc

_N = 100000
_C = 40
_E = 1600000
_M = 50000
_L1 = 10
_A1 = 0.979
_L2 = 10
_A2 = 0.756

_H = _C // 2
_HP = 32
_NSUB = 16
_CH = 16
_EPW = 100352
_EP = _EPW * _NSUB
_EP128 = _EP // 128
_RPW = _EP128 // _NSUB
_OUTER = _RPW // _CH
_NPH = 2
_PH = _N // _NPH
_AGGR = 51200
_ZSTR = _AGGR // _NSUB // 128
_CPW = _PH // _NSUB
_DUMMY = _AGGR - 1

_mesh = plsc.VectorSubcoreMesh(core_axis_name="c", subcore_axis_name="s")


@functools.partial(
    pl.kernel,
    mesh=_mesh,
    compiler_params=pltpu.CompilerParams(use_tc_tiling_on_sc=False),
    out_type=jax.ShapeDtypeStruct((2 * _N, _HP), jnp.float32),
    scratch_types=[
        pltpu.VMEM((128,), jnp.int32),
        pltpu.VMEM((128,), jnp.int32),
        pltpu.VMEM((128, _HP), jnp.float32),
        pltpu.VMEM((128, _HP), jnp.float32),
        pltpu.VMEM_SHARED((_AGGR, _HP), jnp.float32),
        pltpu.SemaphoreType.DMA,
    ],
)
def _seg_sum(z_hbm, srcm_hbm, dstm_hbm, ztile_hbm, out_hbm,
             sidx, didx, rows, zrows, agg, sem):
    c = lax.axis_index("c")
    s = lax.axis_index("s")

    pltpu.sync_copy(ztile_hbm, zrows)
    row0 = c * _EP128 + s * _RPW

    for p in range(_NPH):
        def zbody(j, carry):
            pltpu.sync_copy(zrows,
                            agg.at[pl.ds(s * (_ZSTR * 128) + j * 128, 128)])
            return carry

        lax.fori_loop(0, _ZSTR, zbody, 0)
        plsc.subcore_barrier()

        drow0 = p * _EP128 + s * _RPW

        def ebody(i, carry):
            pltpu.sync_copy(srcm_hbm.at[pl.ds((row0 + i) * 128, 128)], sidx)
            pltpu.sync_copy(dstm_hbm.at[pl.ds((drow0 + i) * 128, 128)], didx)
            pltpu.async_copy(z_hbm.at[sidx], rows, sem).wait()
            pltpu.sync_copy(rows, agg.at[didx], add=True)
            return carry

        lax.fori_loop(0, _RPW, ebody, 0)
        plsc.subcore_barrier()

        base = c * _N + p * _PH + s * _CPW
        pltpu.sync_copy(agg.at[pl.ds(s * _CPW, _CPW)],
                        out_hbm.at[pl.ds(base, _CPW)])

        plsc.subcore_barrier()


_BN = 1000


def _upd_body(agg_ref, h_ref, n_ref, ho_ref, zo_ref, *, alpha, lo, hi):
    a = agg_ref[...]
    h = h_ref[...]
    nb = n_ref[...][None, :, :]
    hn = jnp.clip(alpha * (a * nb) + (1.0 - alpha) * h, lo, hi)
    ho_ref[...] = hn
    zo_ref[...] = hn * nb


def _update(agg, h, norm, alpha, lo, hi):
    body = functools.partial(_upd_body, alpha=alpha, lo=lo, hi=hi)
    return pl.pallas_call(
        body,
        grid=(_N // _BN,),
        in_specs=[
            pl.BlockSpec((2, _BN, _HP), lambda i: (0, i, 0)),
            pl.BlockSpec((2, _BN, _HP), lambda i: (0, i, 0)),
            pl.BlockSpec((_BN, 1), lambda i: (i, 0)),
        ],
        out_specs=[
            pl.BlockSpec((2, _BN, _HP), lambda i: (0, i, 0)),
            pl.BlockSpec((2, _BN, _HP), lambda i: (0, i, 0)),
        ],
        out_shape=[
            jax.ShapeDtypeStruct((2, _N, _HP), jnp.float32),
            jax.ShapeDtypeStruct((2, _N, _HP), jnp.float32),
        ],
    )(agg, h, norm)


def _zmul_body(h_ref, n_ref, zo_ref):
    zo_ref[...] = h_ref[...] * n_ref[...][None, :, :]


def _zmul(h, norm):
    return pl.pallas_call(
        _zmul_body,
        grid=(_N // _BN,),
        in_specs=[
            pl.BlockSpec((2, _BN, _HP), lambda i: (0, i, 0)),
            pl.BlockSpec((_BN, 1), lambda i: (i, 0)),
        ],
        out_specs=pl.BlockSpec((2, _BN, _HP), lambda i: (0, i, 0)),
        out_shape=jax.ShapeDtypeStruct((2, _N, _HP), jnp.float32),
    )(h, norm)


def _norm_body(d_ref, n_ref):
    d = d_ref[...]
    n_ref[...] = jnp.where(d > 0, lax.rsqrt(jnp.maximum(d, 1.0)), 0.0)


def _norm_from_deg(deg):
    return pl.pallas_call(
        _norm_body,
        grid=(_N // _BN,),
        in_specs=[pl.BlockSpec((_BN, 1), lambda i: (i, 0))],
        out_specs=pl.BlockSpec((_BN, 1), lambda i: (i, 0)),
        out_shape=jax.ShapeDtypeStruct((_N, 1), jnp.float32),
    )(deg)


def _to2(x):
    x2 = x.reshape(_N, 2, _H).transpose(1, 0, 2)
    return jnp.pad(x2, ((0, 0), (0, 0), (0, _HP - _H)))


def _from2(x2):
    return x2[:, :, :_H].transpose(1, 0, 2).reshape(_N, _C)


def _propagate(h2, norm, srcm, dstm, ztile, alpha, num_layers, lo, hi):
    z = _zmul(h2, norm)
    for _ in range(num_layers):
        agg = _seg_sum(z.reshape(2 * _N, _HP), srcm, dstm, ztile)
        h2, z = _update(agg.reshape(2, _N, _HP), h2, norm, alpha, lo, hi)
    return h2


def kernel(y_soft, y_true, mask, edge_index):
    src = edge_index[0]
    dst = edge_index[1]

    pad = _EP - _E
    srcp = jnp.concatenate([src, jnp.zeros((pad,), jnp.int32)])
    dstp = jnp.concatenate([dst, jnp.full((pad,), -1, jnp.int32)])
    srcm = jnp.concatenate([srcp, srcp + _N])
    dstl = []
    for p in range(_NPH):
        local = dstp - p * _PH
        ok = (dstp >= p * _PH) & (dstp < (p + 1) * _PH)
        dstl.append(jnp.where(ok, local, _DUMMY))
    dstm = jnp.concatenate(dstl)
    ztile = jnp.zeros((128, _HP), jnp.float32)

    ones2 = jnp.ones((2 * _N, _HP), jnp.float32)
    degagg = _seg_sum(ones2, srcm, dstm, ztile)
    norm = _norm_from_deg(degagg[:_N, :1])

    y_onehot = jax.nn.one_hot(y_true, _C, dtype=y_soft.dtype)

    error = jnp.zeros_like(y_soft).at[mask].set(y_onehot - y_soft[mask])
    se2 = _propagate(_to2(error), norm, srcm, dstm, ztile, _A1, _L1, -1.0, 1.0)
    smoothed_error = _from2(se2)
    sigma = jnp.abs(error[mask]).sum() / float(_M)
    denom = jnp.abs(smoothed_error).sum(axis=1, keepdims=True)
    denom_safe = jnp.where(denom > 0, denom, 1.0)
    scale = sigma / denom_safe
    scale = jnp.where((denom <= 0) | jnp.isinf(scale) | (scale > 1000.0), 1.0, scale)
    result = y_soft + scale * smoothed_error
    result = jnp.where(jnp.isnan(result), y_soft, result)

    yhat = result.at[mask].set(y_onehot)
    out2 = _propagate(_to2(yhat), norm, srcm, dstm, ztile, _A2, _L2, 0.0, 1.0)
    return _from2(out2)
